# Initial kernel scaffold; baseline (speedup 1.0000x reference)
import jax
import jax.numpy as jnp
from jax.experimental import pallas as pl


def kernel(x, Wg, bg, W1, b1, W2, b2):
    raise NotImplementedError("write your pallas kernel here")


# speedup vs baseline: 1.2132x; 1.2132x over previous
import jax
import jax.numpy as jnp
from jax.experimental import pallas as pl
from jax.experimental.pallas import tpu as pltpu

N, D, H, E = 2048, 1024, 2048, 8
NT = 2
TN = N // NT


def _gate_kernel(x_ref, wg_ref, bg_ref, w_ref):
    logits = jnp.dot(x_ref[...], wg_ref[...], preferred_element_type=jnp.float32)
    logits = logits + bg_ref[...]
    eidx = jax.lax.broadcasted_iota(jnp.int32, logits.shape, 1)
    i1 = jnp.argmax(logits, axis=-1)
    v1 = jnp.max(logits, axis=-1)
    masked = jnp.where(eidx == i1[:, None], -jnp.inf, logits)
    i2 = jnp.argmax(masked, axis=-1)
    v2 = jnp.max(masked, axis=-1)
    t = jnp.exp(v2 - v1)
    w1 = 1.0 / (1.0 + t)
    w2 = t / (1.0 + t)
    w_dense = jnp.where(eidx == i1[:, None], w1[:, None],
                        jnp.where(eidx == i2[:, None], w2[:, None], 0.0))
    w_ref[...] = w_dense.astype(jnp.float32).T[:, None, :]


def _moe_kernel(wt_ref, x_ref, w1_ref, b1_ref, w2_ref, b2_ref, out_ref):
    e = pl.program_id(1)

    h = jnp.dot(x_ref[...], w1_ref[0], preferred_element_type=jnp.float32)
    h = jnp.maximum(h + b1_ref[0], 0.0).astype(jnp.bfloat16)
    y = jnp.dot(h, w2_ref[0], preferred_element_type=jnp.float32)
    y = y + b2_ref[0]
    scale = wt_ref[0, 0, :][:, None]
    contrib = scale * y

    @pl.when(e == 0)
    def _():
        out_ref[...] = contrib

    @pl.when(e != 0)
    def _():
        out_ref[...] += contrib


def kernel(x, Wg, bg, W1, b1, W2, b2):
    wT = pl.pallas_call(
        _gate_kernel,
        out_shape=jax.ShapeDtypeStruct((E, 1, N), jnp.float32),
        in_specs=[
            pl.BlockSpec((N, D), lambda: (0, 0)),
            pl.BlockSpec((D, E), lambda: (0, 0)),
            pl.BlockSpec((1, E), lambda: (0, 0)),
        ],
        out_specs=pl.BlockSpec((E, 1, N), lambda: (0, 0, 0)),
    )(x, Wg, bg.reshape(1, E))

    xb = x.astype(jnp.bfloat16)
    W1b = W1.astype(jnp.bfloat16)
    W2b = W2.astype(jnp.bfloat16)

    out = pl.pallas_call(
        _moe_kernel,
        grid=(NT, E),
        out_shape=jax.ShapeDtypeStruct((N, D), jnp.float32),
        in_specs=[
            pl.BlockSpec((1, 1, TN), lambda n, e: (e, 0, n)),
            pl.BlockSpec((TN, D), lambda n, e: (n, 0)),
            pl.BlockSpec((1, D, H), lambda n, e: (e, 0, 0)),
            pl.BlockSpec((1, 1, H), lambda n, e: (e, 0, 0)),
            pl.BlockSpec((1, H, D), lambda n, e: (e, 0, 0)),
            pl.BlockSpec((1, 1, D), lambda n, e: (e, 0, 0)),
        ],
        out_specs=pl.BlockSpec((TN, D), lambda n, e: (n, 0)),
    )(wT, xb, W1b, b1.reshape(E, 1, H), W2b, b2.reshape(E, 1, D))

    return (out, jnp.float32(0.0))


# speedup vs baseline: 1.6144x; 1.3307x over previous
import jax
import jax.numpy as jnp
from jax.experimental import pallas as pl
from jax.experimental.pallas import tpu as pltpu

N, D, H, E = 2048, 1024, 2048, 8
KH = 4
TH = H // KH


def _gate_kernel(x_ref, wg_ref, bg_ref, wcol_ref, wall_ref):
    logits = jnp.dot(x_ref[...], wg_ref[...], preferred_element_type=jnp.float32)
    logits = logits + bg_ref[...]
    eidx = jax.lax.broadcasted_iota(jnp.int32, logits.shape, 1)
    i1 = jnp.argmax(logits, axis=-1)
    v1 = jnp.max(logits, axis=-1)
    masked = jnp.where(eidx == i1[:, None], -jnp.inf, logits)
    i2 = jnp.argmax(masked, axis=-1)
    v2 = jnp.max(masked, axis=-1)
    t = jnp.exp(v2 - v1)
    w1 = 1.0 / (1.0 + t)
    w2 = t / (1.0 + t)
    w_dense = jnp.where(eidx == i1[:, None], w1[:, None],
                        jnp.where(eidx == i2[:, None], w2[:, None], 0.0))
    wall_ref[...] = w_dense
    wcol_ref[...] = w_dense.T[:, :, None]


def _moe_kernel(wcol_ref, wall_ref, x_ref, w1_ref, b1_ref, w2_ref, b2_ref,
                out_ref):
    e = pl.program_id(0)
    kh = pl.program_id(1)

    @pl.when((e == 0) & (kh == 0))
    def _():
        out_ref[...] = jnp.dot(wall_ref[...], b2_ref[...],
                               preferred_element_type=jnp.float32)

    xb = x_ref[...].astype(jnp.bfloat16)
    w1b = w1_ref[0].astype(jnp.bfloat16)
    h = jnp.dot(xb, w1b, preferred_element_type=jnp.float32)
    h = jnp.maximum(h + b1_ref[0], 0.0)
    h = (h * wcol_ref[0]).astype(jnp.bfloat16)
    w2b = w2_ref[0].astype(jnp.bfloat16)
    out_ref[...] += jnp.dot(h, w2b, preferred_element_type=jnp.float32)


def kernel(x, Wg, bg, W1, b1, W2, b2):
    wcol, wall = pl.pallas_call(
        _gate_kernel,
        out_shape=(
            jax.ShapeDtypeStruct((E, N, 1), jnp.float32),
            jax.ShapeDtypeStruct((N, E), jnp.float32),
        ),
        in_specs=[
            pl.BlockSpec((N, D), lambda: (0, 0)),
            pl.BlockSpec((D, E), lambda: (0, 0)),
            pl.BlockSpec((1, E), lambda: (0, 0)),
        ],
        out_specs=(
            pl.BlockSpec((E, N, 1), lambda: (0, 0, 0)),
            pl.BlockSpec((N, E), lambda: (0, 0)),
        ),
    )(x, Wg, bg.reshape(1, E))

    out = pl.pallas_call(
        _moe_kernel,
        grid=(E, KH),
        out_shape=jax.ShapeDtypeStruct((N, D), jnp.float32),
        in_specs=[
            pl.BlockSpec((1, N, 1), lambda e, kh: (e, 0, 0)),
            pl.BlockSpec((N, E), lambda e, kh: (0, 0)),
            pl.BlockSpec((N, D), lambda e, kh: (0, 0)),
            pl.BlockSpec((1, D, TH), lambda e, kh: (e, 0, kh)),
            pl.BlockSpec((1, 1, TH), lambda e, kh: (e, 0, kh)),
            pl.BlockSpec((1, TH, D), lambda e, kh: (e, kh, 0)),
            pl.BlockSpec((E, D), lambda e, kh: (0, 0)),
        ],
        out_specs=pl.BlockSpec((N, D), lambda e, kh: (0, 0)),
    )(wcol, wall, x, W1, b1.reshape(E, 1, H), W2, b2)

    return (out, jnp.float32(0.0))
